# Initial kernel scaffold; baseline (speedup 1.0000x reference)
import jax
import jax.numpy as jnp
from jax.experimental import pallas as pl


def kernel(x, table, W1, b1, W2, b2):
    raise NotImplementedError("write your pallas kernel here")


# speedup vs baseline: 13.0874x; 13.0874x over previous
import functools

import jax
import jax.numpy as jnp
from jax import lax
from jax.experimental import pallas as pl
from jax.experimental.pallas import tpu as pltpu
from jax.experimental.pallas import tpu_sc as plsc

NUM_BUCKETS = 1000000
EMB = 128
LANES = 16
WINDOW = 128

_MULT_LO = 435761
_MULT_HI = 219264


def _hash16(v):
    xh = lax.shift_right_logical(v, jnp.int32(10))
    xl = lax.bitwise_and(v, jnp.int32(1023))
    s = xh * jnp.int32(_MULT_HI) + xl * jnp.int32(_MULT_LO)
    for k in (512, 256, 128, 64, 32, 16, 8, 4, 2, 1):
        t = jnp.int32(k * NUM_BUCKETS)
        s = jnp.where(s >= t, s - t, s)
    return s


def _sc_gather(x32, table):
    n = x32.shape[1]
    mesh = plsc.VectorSubcoreMesh(core_axis_name="core", subcore_axis_name="subcore")

    @functools.partial(
        pl.kernel,
        out_type=jax.ShapeDtypeStruct((n, EMB), jnp.float32),
        mesh=mesh,
        scratch_types=[pltpu.VMEM((WINDOW,), jnp.int32)],
    )
    def gather_kernel(x_hbm, table_hbm, o_hbm, idx_vmem):
        def body(i_vmem, o_vmem):
            for j in range(WINDOW // LANES):
                v = i_vmem[0, pl.ds(j * LANES, LANES)]
                idx_vmem[pl.ds(j * LANES, LANES)] = _hash16(v)
            pltpu.sync_copy(table_hbm.at[idx_vmem], o_vmem)

        pltpu.emit_pipeline(
            body,
            grid=(n // WINDOW,),
            in_specs=[pl.BlockSpec((1, WINDOW), lambda i: (0, i))],
            out_specs=[pl.BlockSpec((WINDOW, EMB), lambda i: (i, 0))],
            core_axis_name=("core", "subcore"),
            dimension_semantics=(pltpu.PARALLEL,),
        )(x_hbm, o_hbm)

    return gather_kernel(x32, table)


def _mlp(embeds, w1t, b1, w2t, b2):
    bsz, k = embeds.shape
    h = w1t.shape[1]
    o = w2t.shape[1]
    bm = 512

    def body(e_ref, w1_ref, b1_ref, w2_ref, b2_ref, o_ref):
        hid = jnp.dot(e_ref[...], w1_ref[...], preferred_element_type=jnp.float32)
        hid = jnp.maximum(hid + b1_ref[...], 0.0)
        o_ref[...] = (
            jnp.dot(hid, w2_ref[...], preferred_element_type=jnp.float32)
            + b2_ref[...]
        )

    return pl.pallas_call(
        body,
        grid=(bsz // bm,),
        in_specs=[
            pl.BlockSpec((bm, k), lambda i: (i, 0)),
            pl.BlockSpec((k, h), lambda i: (0, 0)),
            pl.BlockSpec((1, h), lambda i: (0, 0)),
            pl.BlockSpec((h, o), lambda i: (0, 0)),
            pl.BlockSpec((1, o), lambda i: (0, 0)),
        ],
        out_specs=pl.BlockSpec((bm, o), lambda i: (i, 0)),
        out_shape=jax.ShapeDtypeStruct((bsz, o), jnp.float32),
    )(embeds, w1t, b1, w2t, b2)


def kernel(x, table, W1, b1, W2, b2):
    bsz, nf = x.shape
    x32 = x.astype(jnp.int32).reshape(1, bsz * nf)
    with jax.enable_x64(False):
        embeds = _sc_gather(x32, table)
        e2 = embeds.reshape(bsz, nf * EMB)
        out = _mlp(e2, W1.T, b1.reshape(1, -1), W2.T, b2.reshape(1, -1))
    return out
